# baseline (speedup 1.0000x reference)
import functools

import jax
import jax.numpy as jnp
from jax import lax
from jax.experimental import pallas as pl
from jax.experimental.pallas import tpu as pltpu
from jax.experimental.pallas import tpu_sc as plsc

_NEG = float("-inf")

_L = 16
_CH = 80
_NSEG = 512
_RG = 8
_CG = 4
_TW = 144


def _k1_body(x_ref, w1_ref, b1_ref, w2_ref, batch_ref, lg_ref, m_ref,
             rmax_ref, *, nseg, blk):
    i = pl.program_id(0)
    nb = pl.num_programs(0)

    @pl.when(i == 0)
    def _init():
        rmax_ref[...] = jnp.full((nseg, 1), _NEG, jnp.float32)

    x = x_ref[...]
    h = jnp.dot(x, w1_ref[...], preferred_element_type=jnp.float32)
    h = h + b1_ref[...]
    h = h * jax.nn.sigmoid(h)
    lt = jnp.sum(h * w2_ref[...], axis=1).reshape(1, blk)
    lg_ref[...] = lt.reshape(1, 1, blk)

    bt = batch_ref[0]
    seg = lax.broadcasted_iota(jnp.int32, (nseg, 1), 0)
    oh = bt == seg
    bmax = jnp.max(jnp.where(oh, lt, _NEG), axis=1, keepdims=True)
    nm = jnp.maximum(rmax_ref[...], bmax)
    rmax_ref[...] = nm

    @pl.when(i == nb - 1)
    def _fin():
        m_ref[...] = nm


def _k1(x, W1, b1r, w2r, batch3, nseg, blk, nb, d, h):
    return pl.pallas_call(
        functools.partial(_k1_body, nseg=nseg, blk=blk),
        grid=(nb,),
        in_specs=[
            pl.BlockSpec((blk, d), lambda i: (i, 0)),
            pl.BlockSpec((d, h), lambda i: (0, 0)),
            pl.BlockSpec((1, h), lambda i: (0, 0)),
            pl.BlockSpec((1, h), lambda i: (0, 0)),
            pl.BlockSpec((1, 1, blk), lambda i: (i, 0, 0)),
        ],
        out_specs=[
            pl.BlockSpec((1, 1, blk), lambda i: (i, 0, 0)),
            pl.BlockSpec((nseg, 1), lambda i: (0, 0)),
        ],
        out_shape=[
            jax.ShapeDtypeStruct((nb, 1, blk), jnp.float32),
            jax.ShapeDtypeStruct((nseg, 1), jnp.float32),
        ],
        scratch_shapes=[pltpu.VMEM((nseg, 1), jnp.float32)],
    )(x, W1, b1r, w2r, batch3)


def _k2_body(x_hbm, lg_hbm, b_hbm, m_hbm, out_hbm, xv, lv, bv, ev, mv, tab):
    c = lax.axis_index("c")
    s = lax.axis_index("s")
    wid = c * 16 + s
    rw = wid // _CG
    cg = wid % _CG
    c0 = 156 * rw + jnp.minimum(rw, 2)
    cnt = 156 + (rw < 2).astype(jnp.int32)

    pltpu.sync_copy(m_hbm, mv)

    zero = jnp.zeros((_L,), jnp.float32)

    def zrow(i, carry):
        for k in range(_TW // _L):
            tab[i, pl.ds(k * _L, _L)] = zero
        return carry

    lax.fori_loop(0, _NSEG, zrow, 0)

    def flush(tgt, acc, accd):
        for k in range(8):
            tab[tgt, pl.ds(k * _L, _L)] = tab[tgt, pl.ds(k * _L, _L)] + acc[k]
        tab[tgt, pl.ds(128, _L)] = tab[tgt, pl.ds(128, _L)] + accd

    def chunk_body(ci, carry):
        base = (c0 + ci) * _CH
        pltpu.sync_copy(x_hbm.at[pl.ds(base, _CH), pl.ds(cg * 128, 128)], xv)
        pltpu.sync_copy(lg_hbm.at[pl.ds(base, _CH)], lv)
        pltpu.sync_copy(b_hbm.at[pl.ds(base, _CH)], bv.at[pl.ds(0, _CH)])
        for g in range(_CH // _L):
            b16 = bv[pl.ds(g * _L, _L)]
            l16 = lv[pl.ds(g * _L, _L)]
            m16 = plsc.load_gather(mv, [b16])
            ev[pl.ds(g * _L, _L)] = jnp.exp(l16 - m16)

        def row_body(r, rc):
            racc = rc[:8]
            rd = rc[8]
            rcur = rc[9]
            seg = bv[pl.ds(r, _L)][0]
            is_new = seg != rcur

            @pl.when(is_new)
            def _():
                flush(jnp.maximum(rcur, 0), racc, rd)

            keep = jnp.where(is_new, 0.0, 1.0).astype(jnp.float32)
            e16 = plsc.load_gather(ev, [jnp.full((_L,), r, jnp.int32)])
            newacc = tuple(
                racc[k] * keep + e16 * xv[r, pl.ds(k * _L, _L)]
                for k in range(8))
            return newacc + (rd * keep + e16, seg)

        return lax.fori_loop(0, _CH, row_body, carry)

    carry0 = tuple(zero for _ in range(9)) + (jnp.int32(-1),)
    fc = lax.fori_loop(0, cnt, chunk_body, carry0)
    flush(jnp.maximum(fc[9], 0), fc[:8], fc[8])
    pltpu.sync_copy(tab, out_hbm.at[wid])


def _k2(x, lg, batch, m):
    mesh = plsc.VectorSubcoreMesh(core_axis_name="c", subcore_axis_name="s")
    f = pl.kernel(
        _k2_body,
        out_type=jax.ShapeDtypeStruct((_RG * _CG, _NSEG, _TW), jnp.float32),
        mesh=mesh,
        compiler_params=pltpu.CompilerParams(
            needs_layout_passes=False, use_tc_tiling_on_sc=False),
        scratch_types=[
            pltpu.VMEM((_CH, 128), jnp.float32),
            pltpu.VMEM((_CH,), jnp.float32),
            pltpu.VMEM((_CH + _L,), jnp.int32),
            pltpu.VMEM((_CH,), jnp.float32),
            pltpu.VMEM((_NSEG,), jnp.float32),
            pltpu.VMEM((_NSEG, _TW), jnp.float32),
        ],
    )
    return f(x, lg, batch, m)


def _k3_body(p_ref, out_ref):
    p = p_ref[...]
    psum = jnp.sum(p, axis=0)
    feat = jnp.concatenate([psum[g, :, :128] for g in range(_CG)], axis=1)
    den = psum[0, :, 128:129]
    out_ref[...] = feat / (den + 1e-16)


def _k3(p4):
    return pl.pallas_call(
        _k3_body,
        in_specs=[pl.BlockSpec((_RG, _CG, _NSEG, _TW), lambda: (0, 0, 0, 0))],
        out_specs=pl.BlockSpec((_NSEG, 512), lambda: (0, 0)),
        out_shape=jax.ShapeDtypeStruct((_NSEG, 512), jnp.float32),
    )(p4)


def kernel(x, W1, b1, W2, b2, batch):
    n, d = x.shape
    h = W1.shape[1]
    nseg = _NSEG
    blk = 2000
    nb = n // blk

    batch_i = batch.astype(jnp.int32)
    batch3 = batch_i.reshape(nb, 1, blk)
    b1r = b1.reshape(1, h)
    w2r = W2.reshape(1, h)

    lg3, m = _k1(x, W1, b1r, w2r, batch3, nseg, blk, nb, d, h)
    partials = _k2(x, lg3.reshape(n), batch_i, m.reshape(nseg))
    return _k3(partials.reshape(_RG, _CG, nseg, _TW))


# speedup vs baseline: 1.3032x; 1.3032x over previous
import functools

import jax
import jax.numpy as jnp
from jax import lax
from jax.experimental import pallas as pl
from jax.experimental.pallas import tpu as pltpu
from jax.experimental.pallas import tpu_sc as plsc

_NEG = float("-inf")

_L = 16
_CH = 128
_NSEG = 512
_RG = 8
_CG = 4
_TR = 520


def _k1_body(x_ref, w1_ref, b1_ref, w2_ref, batch_ref, lg_ref, m_ref,
             rmax_ref, *, nseg, blk):
    i = pl.program_id(0)
    nb = pl.num_programs(0)

    @pl.when(i == 0)
    def _init():
        rmax_ref[...] = jnp.full((nseg, 1), _NEG, jnp.float32)

    x = x_ref[...]
    h = jnp.dot(x, w1_ref[...], preferred_element_type=jnp.float32)
    h = h + b1_ref[...]
    h = h * jax.nn.sigmoid(h)
    lt = jnp.sum(h * w2_ref[...], axis=1).reshape(1, blk)
    lg_ref[...] = lt.reshape(1, 1, blk)

    bt = batch_ref[0]
    seg = lax.broadcasted_iota(jnp.int32, (nseg, 1), 0)
    oh = bt == seg
    bmax = jnp.max(jnp.where(oh, lt, _NEG), axis=1, keepdims=True)
    nm = jnp.maximum(rmax_ref[...], bmax)
    rmax_ref[...] = nm

    @pl.when(i == nb - 1)
    def _fin():
        m_ref[...] = nm


def _k1(x, W1, b1r, w2r, batch3, nseg, blk, nb, d, h):
    return pl.pallas_call(
        functools.partial(_k1_body, nseg=nseg, blk=blk),
        grid=(nb,),
        in_specs=[
            pl.BlockSpec((blk, d), lambda i: (i, 0)),
            pl.BlockSpec((d, h), lambda i: (0, 0)),
            pl.BlockSpec((1, h), lambda i: (0, 0)),
            pl.BlockSpec((1, h), lambda i: (0, 0)),
            pl.BlockSpec((1, 1, blk), lambda i: (i, 0, 0)),
        ],
        out_specs=[
            pl.BlockSpec((1, 1, blk), lambda i: (i, 0, 0)),
            pl.BlockSpec((nseg, 1), lambda i: (0, 0)),
        ],
        out_shape=[
            jax.ShapeDtypeStruct((nb, 1, blk), jnp.float32),
            jax.ShapeDtypeStruct((nseg, 1), jnp.float32),
        ],
        scratch_shapes=[pltpu.VMEM((nseg, 1), jnp.float32)],
    )(x, W1, b1r, w2r, batch3)


def _k2_body(x_hbm, lg_hbm, b_hbm, m_hbm, feat_hbm, den_hbm,
             xv, lv, bv, mv, tab, dtab, sems):
    c = lax.axis_index("c")
    s = lax.axis_index("s")
    wid = c * 16 + s
    rw = wid // _CG
    cg = wid % _CG
    c0 = 97 * rw + jnp.minimum(rw, 5)
    cnt = 97 + (rw < 5).astype(jnp.int32)

    pltpu.sync_copy(m_hbm, mv)

    zero = jnp.zeros((_L,), jnp.float32)

    def zrow(i, carry):
        for k in range(128 // _L):
            tab[i, pl.ds(k * _L, _L)] = zero
        return carry

    lax.fori_loop(0, _TR, zrow, 0)
    for i in range(8):
        for k in range(128 // _L):
            dtab[i, pl.ds(k * _L, _L)] = zero

    iota = lax.iota(jnp.int32, _L)
    den_mask = jnp.logical_and(iota == 0, jnp.full((_L,), cg == 0))
    col_base = cg * 128

    def fire(ci, b):
        base = ci * _CH
        pltpu.async_copy(
            x_hbm.at[pl.ds(base, _CH), pl.ds(col_base, 128)], xv.at[b],
            sems.at[b])
        pltpu.async_copy(lg_hbm.at[pl.ds(base, _CH)], lv.at[b], sems.at[b])
        pltpu.async_copy(b_hbm.at[pl.ds(base, _CH)], bv.at[b], sems.at[b])

    def drain(ci, b):
        base = ci * _CH
        pltpu.make_async_copy(
            x_hbm.at[pl.ds(base, _CH), pl.ds(col_base, 128)], xv.at[b],
            sems.at[b]).wait()
        pltpu.make_async_copy(
            lg_hbm.at[pl.ds(base, _CH)], lv.at[b], sems.at[b]).wait()
        pltpu.make_async_copy(
            b_hbm.at[pl.ds(base, _CH)], bv.at[b], sems.at[b]).wait()

    fire(c0, 0)

    def chunk_body(i, carry):
        ci = c0 + i
        b = lax.rem(i, 2)

        @pl.when(i + 1 < cnt)
        def _():
            fire(ci + 1, 1 - b)

        drain(ci, b)

        lax.fori_loop(0, _CH // _L, lambda g, gc: _group(
            xv, lv, bv, mv, tab, dtab, iota, den_mask, b, g) or gc, 0)
        return carry

    lax.fori_loop(0, cnt, chunk_body, 0)

    @pl.when(rw == 7)
    def _tail():
        base = 100000 - _CH
        pltpu.sync_copy(
            x_hbm.at[pl.ds(base, _CH), pl.ds(col_base, 128)], xv.at[0])
        pltpu.sync_copy(lg_hbm.at[pl.ds(base, _CH)], lv.at[0])
        pltpu.sync_copy(b_hbm.at[pl.ds(base, _CH)], bv.at[0])
        for g in range(6, 8):
            _group(xv, lv, bv, mv, tab, dtab, iota, den_mask, 0, g)

    pltpu.sync_copy(tab, feat_hbm.at[wid])
    pltpu.sync_copy(dtab, den_hbm.at[wid])


def _group(xv, lv, bv, mv, tab, dtab, iota, den_mask, b, g):
    b16 = bv[b, pl.ds(g * _L, _L)]
    l16 = lv[b, pl.ds(g * _L, _L)]
    m16 = plsc.load_gather(mv, [b16])
    e16 = jnp.exp(l16 - m16)
    for r in range(_L):
        seg = b16[r]
        e_b = e16[r]
        rowidx = jnp.full((_L,), seg, jnp.int32)
        for k in range(128 // _L):
            xvk = xv[b, g * _L + r, pl.ds(k * _L, _L)]
            plsc.addupdate_scatter(
                tab, [rowidx, iota + (k * _L)], e_b * xvk)
        plsc.addupdate_scatter(
            dtab,
            [jnp.full((_L,), lax.shift_right_logical(seg, 7), jnp.int32),
             jnp.full((_L,), lax.bitwise_and(seg, 127), jnp.int32)],
            jnp.full((_L,), e_b, jnp.float32), mask=den_mask)


def _k2(x, lg, batch, m):
    mesh = plsc.VectorSubcoreMesh(core_axis_name="c", subcore_axis_name="s")
    f = pl.kernel(
        _k2_body,
        out_type=[
            jax.ShapeDtypeStruct((_RG * _CG, _TR, 128), jnp.float32),
            jax.ShapeDtypeStruct((_RG * _CG, 8, 128), jnp.float32),
        ],
        mesh=mesh,
        compiler_params=pltpu.CompilerParams(needs_layout_passes=False),
        scratch_types=[
            pltpu.VMEM((2, _CH, 128), jnp.float32),
            pltpu.VMEM((2, _CH), jnp.float32),
            pltpu.VMEM((2, _CH), jnp.int32),
            pltpu.VMEM((_NSEG,), jnp.float32),
            pltpu.VMEM((_TR, 128), jnp.float32),
            pltpu.VMEM((8, 128), jnp.float32),
            pltpu.SemaphoreType.DMA((2,)),
        ],
    )
    return f(x, lg, batch, m)


def _k3_body(p_ref, d_ref, out_ref):
    p = p_ref[...][:, :, :_NSEG, :]
    psum = jnp.sum(p, axis=0)
    feat = jnp.concatenate([psum[g] for g in range(_CG)], axis=1)
    dsum = jnp.sum(d_ref[...], axis=0)
    den = jnp.concatenate(
        [dsum[i].reshape(128, 1) for i in range(4)], axis=0)
    out_ref[...] = feat / (den + 1e-16)


def _k3(p4, d4):
    return pl.pallas_call(
        _k3_body,
        in_specs=[
            pl.BlockSpec((_RG, _CG, _TR, 128), lambda: (0, 0, 0, 0)),
            pl.BlockSpec((_RG * _CG, 8, 128), lambda: (0, 0, 0)),
        ],
        out_specs=pl.BlockSpec((_NSEG, 512), lambda: (0, 0)),
        out_shape=jax.ShapeDtypeStruct((_NSEG, 512), jnp.float32),
    )(p4, d4)


def kernel(x, W1, b1, W2, b2, batch):
    n, d = x.shape
    h = W1.shape[1]
    nseg = _NSEG
    blk = 2000
    nb = n // blk

    batch_i = batch.astype(jnp.int32)
    batch3 = batch_i.reshape(nb, 1, blk)
    b1r = b1.reshape(1, h)
    w2r = W2.reshape(1, h)

    lg3, m = _k1(x, W1, b1r, w2r, batch3, nseg, blk, nb, d, h)
    feat, den = _k2(x, lg3.reshape(n), batch_i, m.reshape(nseg))
    return _k3(feat.reshape(_RG, _CG, _TR, 128), den)


# speedup vs baseline: 2.6967x; 2.0693x over previous
import functools

import jax
import jax.numpy as jnp
from jax import lax
from jax.experimental import pallas as pl
from jax.experimental.pallas import tpu as pltpu
from jax.experimental.pallas import tpu_sc as plsc

_NEG = float("-inf")

_N = 100000
_L = 16
_CH = 128
_NSEG = 512
_RG = 8
_CG = 4
_TR = 520


def _k1_body(x_ref, w1_ref, b1_ref, w2_ref, batch_ref, lg_ref, m_ref,
             rmax_ref, *, nseg, blk):
    i = pl.program_id(0)
    nb = pl.num_programs(0)

    @pl.when(i == 0)
    def _init():
        rmax_ref[...] = jnp.full((nseg, 1), _NEG, jnp.float32)

    x = x_ref[...]
    h = jnp.dot(x, w1_ref[...], preferred_element_type=jnp.float32)
    h = h + b1_ref[...]
    h = h * jax.nn.sigmoid(h)
    lt = jnp.sum(h * w2_ref[...], axis=1).reshape(1, blk)
    lg_ref[...] = lt.reshape(1, 1, blk)

    bt = batch_ref[0]
    seg = lax.broadcasted_iota(jnp.int32, (nseg, 1), 0)
    oh = bt == seg
    bmax = jnp.max(jnp.where(oh, lt, _NEG), axis=1, keepdims=True)
    nm = jnp.maximum(rmax_ref[...], bmax)
    rmax_ref[...] = nm

    @pl.when(i == nb - 1)
    def _fin():
        m_ref[...] = nm


def _k1(x, W1, b1r, w2r, batch3, nseg, blk, nb, d, h):
    return pl.pallas_call(
        functools.partial(_k1_body, nseg=nseg, blk=blk),
        grid=(nb,),
        in_specs=[
            pl.BlockSpec((blk, d), lambda i: (i, 0)),
            pl.BlockSpec((d, h), lambda i: (0, 0)),
            pl.BlockSpec((1, h), lambda i: (0, 0)),
            pl.BlockSpec((1, h), lambda i: (0, 0)),
            pl.BlockSpec((1, 1, blk), lambda i: (i, 0, 0)),
        ],
        out_specs=[
            pl.BlockSpec((1, 1, blk), lambda i: (i, 0, 0)),
            pl.BlockSpec((nseg, 1), lambda i: (0, 0)),
        ],
        out_shape=[
            jax.ShapeDtypeStruct((nb, 1, blk), jnp.float32),
            jax.ShapeDtypeStruct((nseg, 1), jnp.float32),
        ],
        scratch_shapes=[pltpu.VMEM((nseg, 1), jnp.float32)],
    )(x, W1, b1r, w2r, batch3)


def _k2_body(x_hbm, lg_hbm, b_hbm, m_hbm, feat_hbm, den_hbm,
             xv, lv, bv, mv, tab, dtab, sems):
    c = lax.axis_index("c")
    s = lax.axis_index("s")
    wid = c * 16 + s
    rw = wid // _CG
    cg = wid % _CG
    c0 = 97 * rw + jnp.minimum(rw, 5)
    cnt = 97 + (rw < 5).astype(jnp.int32) + (rw == 7).astype(jnp.int32)

    pltpu.sync_copy(m_hbm, mv)

    zero = jnp.zeros((_L,), jnp.float32)

    def zrow(i, carry):
        for k in range(128 // _L):
            tab[i, pl.ds(k * _L, _L)] = zero
        return carry

    lax.fori_loop(0, _TR, zrow, 0)

    def zdrow(i, carry):
        for k in range(128 // _L):
            dtab[i, pl.ds(k * _L, _L)] = zero
        return carry

    lax.fori_loop(0, 64, zdrow, 0)

    iota = lax.iota(jnp.int32, _L)
    lane0 = (iota == 0).astype(jnp.float32)
    col_base = cg * 128

    def chunk_base(ci):
        return jnp.minimum(ci * _CH, _N - _CH)

    def fire(ci, b):
        base = chunk_base(ci)
        pltpu.async_copy(
            x_hbm.at[pl.ds(base, _CH), pl.ds(col_base, 128)], xv.at[b],
            sems.at[b])
        pltpu.async_copy(lg_hbm.at[pl.ds(base, _CH)], lv.at[b], sems.at[b])
        pltpu.async_copy(b_hbm.at[pl.ds(base, _CH)], bv.at[b], sems.at[b])

    def drain(ci, b):
        base = chunk_base(ci)
        pltpu.make_async_copy(
            x_hbm.at[pl.ds(base, _CH), pl.ds(col_base, 128)], xv.at[b],
            sems.at[b]).wait()
        pltpu.make_async_copy(
            lg_hbm.at[pl.ds(base, _CH)], lv.at[b], sems.at[b]).wait()
        pltpu.make_async_copy(
            b_hbm.at[pl.ds(base, _CH)], bv.at[b], sems.at[b]).wait()

    def flush(tgt, acc, accd):
        for k in range(128 // _L):
            tab[tgt, pl.ds(k * _L, _L)] = tab[tgt, pl.ds(k * _L, _L)] + acc[k]
        plsc.addupdate_scatter(
            dtab,
            [jnp.full((_L,), lax.shift_right_logical(tgt, 3), jnp.int32),
             jnp.full((_L,), lax.bitwise_and(tgt, 7) * _L, jnp.int32) + iota],
            accd)

    fire(c0, 0)

    def chunk_body(i, carry):
        ci = c0 + i
        b = lax.rem(i, 2)

        @pl.when(i + 1 < cnt)
        def _():
            fire(ci + 1, 1 - b)

        drain(ci, b)
        glo = jnp.where(ci * _CH > _N - _CH, (_CH - 32) // _L, 0)

        def group_body(g, gc):
            acc = gc[:8]
            accd = gc[8]
            cur = gc[9]
            b16 = bv[b, pl.ds(g * _L, _L)]
            l16 = lv[b, pl.ds(g * _L, _L)]
            m16 = plsc.load_gather(mv, [b16])
            e16 = jnp.exp(l16 - m16)
            seg0 = b16[0]

            def fast(*op):
                facc = list(op[:8])
                faccd = op[8]
                fcur = op[9]

                @pl.when(seg0 != fcur)
                def _():
                    flush(jnp.maximum(fcur, 0), facc, faccd)

                keep = jnp.where(seg0 == fcur, 1.0, 0.0)
                facc = [a * keep for a in facc]
                faccd = faccd * keep + e16
                for r in range(_L):
                    e_b = e16[r]
                    for k in range(128 // _L):
                        facc[k] = facc[k] + e_b * xv[
                            b, g * _L + r, pl.ds(k * _L, _L)]
                return tuple(facc) + (faccd, seg0)

            def slow(*op):
                sacc = list(op[:8])
                saccd = op[8]
                scur = op[9]
                for r in range(_L):
                    seg = b16[r]
                    e_b = e16[r]

                    @pl.when(seg != scur)
                    def _(sacc=sacc, saccd=saccd, scur=scur):
                        flush(jnp.maximum(scur, 0), sacc, saccd)

                    keep = jnp.where(seg == scur, 1.0, 0.0)
                    for k in range(128 // _L):
                        sacc[k] = sacc[k] * keep + e_b * xv[
                            b, g * _L + r, pl.ds(k * _L, _L)]
                    saccd = saccd * keep + e_b * lane0
                    scur = seg
                return tuple(sacc) + (saccd, scur)

            return lax.cond(seg0 == b16[_L - 1], fast, slow, *gc)

        return lax.fori_loop(glo, _CH // _L, group_body, carry)

    carry0 = tuple(zero for _ in range(9)) + (jnp.int32(-1),)
    fc = lax.fori_loop(0, cnt, chunk_body, carry0)
    flush(jnp.maximum(fc[9], 0), list(fc[:8]), fc[8])

    pltpu.sync_copy(tab, feat_hbm.at[wid])
    pltpu.sync_copy(dtab, den_hbm.at[wid])


def _k2(x, lg, batch, m):
    mesh = plsc.VectorSubcoreMesh(core_axis_name="c", subcore_axis_name="s")
    f = pl.kernel(
        _k2_body,
        out_type=[
            jax.ShapeDtypeStruct((_RG * _CG, _TR, 128), jnp.float32),
            jax.ShapeDtypeStruct((_RG * _CG, 64, 128), jnp.float32),
        ],
        mesh=mesh,
        compiler_params=pltpu.CompilerParams(needs_layout_passes=False),
        scratch_types=[
            pltpu.VMEM((2, _CH, 128), jnp.float32),
            pltpu.VMEM((2, _CH), jnp.float32),
            pltpu.VMEM((2, _CH), jnp.int32),
            pltpu.VMEM((_NSEG,), jnp.float32),
            pltpu.VMEM((_TR, 128), jnp.float32),
            pltpu.VMEM((64, 128), jnp.float32),
            pltpu.SemaphoreType.DMA((2,)),
        ],
    )
    return f(x, lg, batch, m)


def _k3_body(p_ref, d_ref, out_ref):
    p = p_ref[...][:, :, :_NSEG, :]
    psum = jnp.sum(p, axis=0)
    feat = jnp.concatenate([psum[g] for g in range(_CG)], axis=1)
    dsum = jnp.sum(d_ref[...], axis=0)
    srow = lax.broadcasted_iota(jnp.int32, (_NSEG, 64), 0)
    rcol = lax.broadcasted_iota(jnp.int32, (_NSEG, 64), 1)
    sel = (rcol == lax.shift_right_logical(srow, 3)).astype(jnp.float32)
    g = jnp.dot(sel, dsum, preferred_element_type=jnp.float32)
    sc = lax.broadcasted_iota(jnp.int32, (_NSEG, 128), 0)
    cc = lax.broadcasted_iota(jnp.int32, (_NSEG, 128), 1)
    win = (lax.shift_right_logical(cc, 4) ==
           lax.bitwise_and(sc, 7)).astype(jnp.float32)
    den = jnp.sum(g * win, axis=1, keepdims=True) * 0.25
    out_ref[...] = feat / (den + 1e-16)


def _k3(p4, d4):
    return pl.pallas_call(
        _k3_body,
        in_specs=[
            pl.BlockSpec((_RG, _CG, _TR, 128), lambda: (0, 0, 0, 0)),
            pl.BlockSpec((_RG * _CG, 64, 128), lambda: (0, 0, 0)),
        ],
        out_specs=pl.BlockSpec((_NSEG, 512), lambda: (0, 0)),
        out_shape=jax.ShapeDtypeStruct((_NSEG, 512), jnp.float32),
    )(p4, d4)


def kernel(x, W1, b1, W2, b2, batch):
    n, d = x.shape
    h = W1.shape[1]
    nseg = _NSEG
    blk = 2000
    nb = n // blk

    batch_i = batch.astype(jnp.int32)
    batch3 = batch_i.reshape(nb, 1, blk)
    b1r = b1.reshape(1, h)
    w2r = W2.reshape(1, h)

    lg3, m = _k1(x, W1, b1r, w2r, batch3, nseg, blk, nb, d, h)
    feat, den = _k2(x, lg3.reshape(n), batch_i, m.reshape(nseg))
    return _k3(feat.reshape(_RG, _CG, _TR, 128), den)


# speedup vs baseline: 2.8968x; 1.0742x over previous
import functools

import jax
import jax.numpy as jnp
from jax import lax
from jax.experimental import pallas as pl
from jax.experimental.pallas import tpu as pltpu
from jax.experimental.pallas import tpu_sc as plsc

_NEG = float("-inf")

_N = 100000
_L = 16
_CH = 128
_NSEG = 512
_RG = 8
_CG = 4
_TR = 520


def _k1_body(x_ref, w1_ref, b1_ref, w2_ref, batch_ref, lg_ref, m_ref,
             rmax_ref, *, nseg, blk):
    i = pl.program_id(0)
    nb = pl.num_programs(0)

    @pl.when(i == 0)
    def _init():
        rmax_ref[...] = jnp.full((1, nseg), _NEG, jnp.bfloat16)

    x = x_ref[...]
    h = jnp.dot(x, w1_ref[...], preferred_element_type=jnp.float32)
    h = h + b1_ref[...]
    h = h * jax.nn.sigmoid(h)
    lt = jnp.dot(h, w2_ref[...], preferred_element_type=jnp.float32)
    lg_ref[...] = lt.reshape(1, blk, 1)

    bt = batch_ref[0].astype(jnp.int16)
    seg = lax.broadcasted_iota(jnp.int16, (1, nseg), 1)
    oh = bt == seg
    ltb = lt.astype(jnp.bfloat16)
    bmax = jnp.max(jnp.where(oh, ltb, jnp.bfloat16(_NEG)),
                   axis=0, keepdims=True)
    nm = jnp.maximum(rmax_ref[...], bmax)
    rmax_ref[...] = nm

    @pl.when(i == nb - 1)
    def _fin():
        m_ref[...] = nm.astype(jnp.float32)


def _k1(x, W1, b1r, w2r, batch3, nseg, blk, nb, d, h):
    return pl.pallas_call(
        functools.partial(_k1_body, nseg=nseg, blk=blk),
        grid=(nb,),
        in_specs=[
            pl.BlockSpec((blk, d), lambda i: (i, 0)),
            pl.BlockSpec((d, h), lambda i: (0, 0)),
            pl.BlockSpec((1, h), lambda i: (0, 0)),
            pl.BlockSpec((h, 1), lambda i: (0, 0)),
            pl.BlockSpec((1, blk, 1), lambda i: (i, 0, 0)),
        ],
        out_specs=[
            pl.BlockSpec((1, blk, 1), lambda i: (i, 0, 0)),
            pl.BlockSpec((1, nseg), lambda i: (0, 0)),
        ],
        out_shape=[
            jax.ShapeDtypeStruct((nb, blk, 1), jnp.float32),
            jax.ShapeDtypeStruct((1, nseg), jnp.float32),
        ],
        scratch_shapes=[pltpu.VMEM((1, nseg), jnp.bfloat16)],
    )(x, W1, b1r, w2r, batch3)


def _k2_body(x_hbm, lg_hbm, b_hbm, m_hbm, feat_hbm, den_hbm,
             xv, lv, bv, mv, tab, dtab, sems):
    c = lax.axis_index("c")
    s = lax.axis_index("s")
    wid = c * 16 + s
    rw = wid // _CG
    cg = wid % _CG
    c0 = 97 * rw + jnp.minimum(rw, 5)
    cnt = 97 + (rw < 5).astype(jnp.int32) + (rw == 7).astype(jnp.int32)

    pltpu.sync_copy(m_hbm, mv)

    zero = jnp.zeros((_L,), jnp.float32)

    def zrow(i, carry):
        for k in range(128 // _L):
            tab[i, pl.ds(k * _L, _L)] = zero
        return carry

    lax.fori_loop(0, _TR, zrow, 0)

    def zdrow(i, carry):
        for k in range(128 // _L):
            dtab[i, pl.ds(k * _L, _L)] = zero
        return carry

    lax.fori_loop(0, 64, zdrow, 0)

    iota = lax.iota(jnp.int32, _L)
    lane0 = (iota == 0).astype(jnp.float32)
    col_base = cg * 128

    def chunk_base(ci):
        return jnp.minimum(ci * _CH, _N - _CH)

    def fire(ci, b):
        base = chunk_base(ci)
        pltpu.async_copy(
            x_hbm.at[pl.ds(base, _CH), pl.ds(col_base, 128)], xv.at[b],
            sems.at[b])
        pltpu.async_copy(lg_hbm.at[pl.ds(base, _CH)], lv.at[b], sems.at[b])
        pltpu.async_copy(b_hbm.at[pl.ds(base, _CH)], bv.at[b], sems.at[b])

    def drain(ci, b):
        base = chunk_base(ci)
        pltpu.make_async_copy(
            x_hbm.at[pl.ds(base, _CH), pl.ds(col_base, 128)], xv.at[b],
            sems.at[b]).wait()
        pltpu.make_async_copy(
            lg_hbm.at[pl.ds(base, _CH)], lv.at[b], sems.at[b]).wait()
        pltpu.make_async_copy(
            b_hbm.at[pl.ds(base, _CH)], bv.at[b], sems.at[b]).wait()

    def flush(tgt, acc, accd):
        for k in range(128 // _L):
            tab[tgt, pl.ds(k * _L, _L)] = tab[tgt, pl.ds(k * _L, _L)] + acc[k]
        plsc.addupdate_scatter(
            dtab,
            [jnp.full((_L,), lax.shift_right_logical(tgt, 3), jnp.int32),
             jnp.full((_L,), lax.bitwise_and(tgt, 7) * _L, jnp.int32) + iota],
            accd)

    fire(c0, 0)

    def chunk_body(i, carry):
        ci = c0 + i
        b = lax.rem(i, 2)

        @pl.when(i + 1 < cnt)
        def _():
            fire(ci + 1, 1 - b)

        drain(ci, b)
        glo = jnp.where(ci * _CH > _N - _CH, (_CH - 32) // _L, 0)

        def group_body(g, gc):
            acc = gc[:8]
            accd = gc[8]
            cur = gc[9]
            b16 = bv[b, pl.ds(g * _L, _L)]
            l16 = lv[b, pl.ds(g * _L, _L)]
            m16 = plsc.load_gather(mv, [b16])
            e16 = jnp.exp(l16 - m16)
            seg0 = b16[0]

            def fast(*op):
                facc = list(op[:8])
                faccd = op[8]
                fcur = op[9]

                @pl.when(seg0 != fcur)
                def _():
                    flush(jnp.maximum(fcur, 0), facc, faccd)

                keep = jnp.where(seg0 == fcur, 1.0, 0.0)
                facc = [a * keep for a in facc]
                faccd = faccd * keep + e16
                for r in range(_L):
                    e_b = e16[r]
                    for k in range(128 // _L):
                        facc[k] = facc[k] + e_b * xv[
                            b, g * _L + r, pl.ds(k * _L, _L)]
                return tuple(facc) + (faccd, seg0)

            def slow(*op):
                sacc = list(op[:8])
                saccd = op[8]
                scur = op[9]
                for r in range(_L):
                    seg = b16[r]
                    e_b = e16[r]

                    @pl.when(seg != scur)
                    def _(sacc=sacc, saccd=saccd, scur=scur):
                        flush(jnp.maximum(scur, 0), sacc, saccd)

                    keep = jnp.where(seg == scur, 1.0, 0.0)
                    for k in range(128 // _L):
                        sacc[k] = sacc[k] * keep + e_b * xv[
                            b, g * _L + r, pl.ds(k * _L, _L)]
                    saccd = saccd * keep + e_b * lane0
                    scur = seg
                return tuple(sacc) + (saccd, scur)

            return lax.cond(seg0 == b16[_L - 1], fast, slow, *gc)

        return lax.fori_loop(glo, _CH // _L, group_body, carry)

    carry0 = tuple(zero for _ in range(9)) + (jnp.int32(-1),)
    fc = lax.fori_loop(0, cnt, chunk_body, carry0)
    flush(jnp.maximum(fc[9], 0), list(fc[:8]), fc[8])

    pltpu.sync_copy(tab, feat_hbm.at[wid])
    pltpu.sync_copy(dtab, den_hbm.at[wid])


def _k2(x, lg, batch, m):
    mesh = plsc.VectorSubcoreMesh(core_axis_name="c", subcore_axis_name="s")
    f = pl.kernel(
        _k2_body,
        out_type=[
            jax.ShapeDtypeStruct((_RG * _CG, _TR, 128), jnp.float32),
            jax.ShapeDtypeStruct((_RG * _CG, 64, 128), jnp.float32),
        ],
        mesh=mesh,
        compiler_params=pltpu.CompilerParams(needs_layout_passes=False),
        scratch_types=[
            pltpu.VMEM((2, _CH, 128), jnp.float32),
            pltpu.VMEM((2, _CH), jnp.float32),
            pltpu.VMEM((2, _CH), jnp.int32),
            pltpu.VMEM((_NSEG,), jnp.float32),
            pltpu.VMEM((_TR, 128), jnp.float32),
            pltpu.VMEM((64, 128), jnp.float32),
            pltpu.SemaphoreType.DMA((2,)),
        ],
    )
    return f(x, lg, batch, m)


def _k3_body(p_ref, d_ref, out_ref):
    p = p_ref[...][:, :, :_NSEG, :]
    psum = jnp.sum(p, axis=0)
    feat = jnp.concatenate([psum[g] for g in range(_CG)], axis=1)
    dsum = jnp.sum(d_ref[...], axis=0)
    srow = lax.broadcasted_iota(jnp.int32, (_NSEG, 64), 0)
    rcol = lax.broadcasted_iota(jnp.int32, (_NSEG, 64), 1)
    sel = (rcol == lax.shift_right_logical(srow, 3)).astype(jnp.float32)
    g = jnp.dot(sel, dsum, preferred_element_type=jnp.float32)
    sc = lax.broadcasted_iota(jnp.int32, (_NSEG, 128), 0)
    cc = lax.broadcasted_iota(jnp.int32, (_NSEG, 128), 1)
    win = (lax.shift_right_logical(cc, 4) ==
           lax.bitwise_and(sc, 7)).astype(jnp.float32)
    den = jnp.sum(g * win, axis=1, keepdims=True) * 0.25
    out_ref[...] = feat / (den + 1e-16)


def _k3(p4, d4):
    return pl.pallas_call(
        _k3_body,
        in_specs=[
            pl.BlockSpec((_RG, _CG, _TR, 128), lambda: (0, 0, 0, 0)),
            pl.BlockSpec((_RG * _CG, 64, 128), lambda: (0, 0, 0)),
        ],
        out_specs=pl.BlockSpec((_NSEG, 512), lambda: (0, 0)),
        out_shape=jax.ShapeDtypeStruct((_NSEG, 512), jnp.float32),
    )(p4, d4)


def kernel(x, W1, b1, W2, b2, batch):
    n, d = x.shape
    h = W1.shape[1]
    nseg = _NSEG
    blk = 4000
    nb = n // blk

    batch_i = batch.astype(jnp.int32)
    batch3 = batch_i.reshape(nb, blk, 1)
    b1r = b1.reshape(1, h)

    lg3, m = _k1(x, W1, b1r, W2, batch3, nseg, blk, nb, d, h)
    feat, den = _k2(x, lg3.reshape(n), batch_i, m.reshape(nseg))
    return _k3(feat.reshape(_RG, _CG, _TR, 128), den)


# speedup vs baseline: 3.9319x; 1.3573x over previous
import functools

import jax
import jax.numpy as jnp
from jax import lax
from jax.experimental import pallas as pl
from jax.experimental.pallas import tpu as pltpu
from jax.experimental.pallas import tpu_sc as plsc

_NEG = float("-inf")

_N = 100000
_L = 16
_CH = 128
_NSEG = 512
_RG = 8
_CG = 4
_TR = 520


def _k1_body(x_ref, w1_ref, b1_ref, w2_ref, batch_ref, lg_ref, m_ref,
             rmax_ref, *, nseg, blk):
    i = pl.program_id(0)
    nb = pl.num_programs(0)

    @pl.when(i == 0)
    def _init():
        rmax_ref[...] = jnp.full((nseg, 1), _NEG, jnp.bfloat16)

    x = x_ref[...]
    h = jnp.dot(x, w1_ref[...], preferred_element_type=jnp.float32)
    h = h + b1_ref[...]
    h = h * jax.nn.sigmoid(h)
    lt_row = lax.dot_general(w2_ref[...], h, (((0,), (1,)), ((), ())),
                             preferred_element_type=jnp.float32)
    lg_ref[...] = lt_row.reshape(1, 1, blk)

    bt = batch_ref[0].astype(jnp.int16)
    seg = lax.broadcasted_iota(jnp.int16, (nseg, 1), 0)
    oh = bt == seg
    ltb = lt_row.astype(jnp.bfloat16)
    bmax = jnp.max(jnp.where(oh, ltb, jnp.bfloat16(_NEG)),
                   axis=1, keepdims=True)
    nm = jnp.maximum(rmax_ref[...], bmax)
    rmax_ref[...] = nm

    @pl.when(i == nb - 1)
    def _fin():
        m_ref[...] = nm.astype(jnp.float32)


def _k1(x, W1, b1r, w2r, batch3, nseg, blk, nb, d, h):
    return pl.pallas_call(
        functools.partial(_k1_body, nseg=nseg, blk=blk),
        grid=(nb,),
        in_specs=[
            pl.BlockSpec((blk, d), lambda i: (i, 0)),
            pl.BlockSpec((d, h), lambda i: (0, 0)),
            pl.BlockSpec((1, h), lambda i: (0, 0)),
            pl.BlockSpec((h, 1), lambda i: (0, 0)),
            pl.BlockSpec((1, 1, blk), lambda i: (i, 0, 0)),
        ],
        out_specs=[
            pl.BlockSpec((1, 1, blk), lambda i: (i, 0, 0)),
            pl.BlockSpec((nseg, 1), lambda i: (0, 0)),
        ],
        out_shape=[
            jax.ShapeDtypeStruct((nb, 1, blk), jnp.float32),
            jax.ShapeDtypeStruct((nseg, 1), jnp.float32),
        ],
        scratch_shapes=[pltpu.VMEM((nseg, 1), jnp.bfloat16)],
    )(x, W1, b1r, w2r, batch3)


def _k2_body(x_hbm, lg_hbm, b_hbm, m_hbm, feat_hbm, den_hbm,
             xv, lv, bv, mv, tab, dtab, sems):
    c = lax.axis_index("c")
    s = lax.axis_index("s")
    wid = c * 16 + s
    rw = wid // _CG
    cg = wid % _CG
    c0 = 97 * rw + jnp.minimum(rw, 5)
    cnt = 97 + (rw < 5).astype(jnp.int32) + (rw == 7).astype(jnp.int32)

    pltpu.sync_copy(m_hbm, mv)

    zero = jnp.zeros((_L,), jnp.float32)

    def zrow(i, carry):
        for k in range(128 // _L):
            tab[i, pl.ds(k * _L, _L)] = zero
        return carry

    lax.fori_loop(0, _TR, zrow, 0)

    def zdrow(i, carry):
        for k in range(128 // _L):
            dtab[i, pl.ds(k * _L, _L)] = zero
        return carry

    lax.fori_loop(0, 64, zdrow, 0)

    iota = lax.iota(jnp.int32, _L)
    lane0 = (iota == 0).astype(jnp.float32)
    col_base = cg * 128

    def chunk_base(ci):
        return jnp.minimum(ci * _CH, _N - _CH)

    def fire(ci, b):
        base = chunk_base(ci)
        pltpu.async_copy(
            x_hbm.at[pl.ds(base, _CH), pl.ds(col_base, 128)], xv.at[b],
            sems.at[b])
        pltpu.async_copy(lg_hbm.at[pl.ds(base, _CH)], lv.at[b], sems.at[b])
        pltpu.async_copy(b_hbm.at[pl.ds(base, _CH)], bv.at[b], sems.at[b])

    def drain(ci, b):
        base = chunk_base(ci)
        pltpu.make_async_copy(
            x_hbm.at[pl.ds(base, _CH), pl.ds(col_base, 128)], xv.at[b],
            sems.at[b]).wait()
        pltpu.make_async_copy(
            lg_hbm.at[pl.ds(base, _CH)], lv.at[b], sems.at[b]).wait()
        pltpu.make_async_copy(
            b_hbm.at[pl.ds(base, _CH)], bv.at[b], sems.at[b]).wait()

    def flush(tgt, acc, accd):
        for k in range(128 // _L):
            tab[tgt, pl.ds(k * _L, _L)] = tab[tgt, pl.ds(k * _L, _L)] + acc[k]
        plsc.addupdate_scatter(
            dtab,
            [jnp.full((_L,), lax.shift_right_logical(tgt, 3), jnp.int32),
             jnp.full((_L,), lax.bitwise_and(tgt, 7) * _L, jnp.int32) + iota],
            accd)

    fire(c0, 0)

    def chunk_body(i, carry):
        ci = c0 + i
        b = lax.rem(i, 2)

        @pl.when(i + 1 < cnt)
        def _():
            fire(ci + 1, 1 - b)

        drain(ci, b)
        glo = jnp.where(ci * _CH > _N - _CH, (_CH - 32) // _L, 0)

        def group_body(g, gc):
            acc = gc[:8]
            accd = gc[8]
            cur = gc[9]
            b16 = bv[b, pl.ds(g * _L, _L)]
            l16 = lv[b, pl.ds(g * _L, _L)]
            m16 = plsc.load_gather(mv, [b16])
            e16 = jnp.exp(l16 - m16)
            seg0 = b16[0]

            def fast(*op):
                facc = list(op[:8])
                faccd = op[8]
                fcur = op[9]

                @pl.when(seg0 != fcur)
                def _():
                    flush(jnp.maximum(fcur, 0), facc, faccd)

                keep = jnp.where(seg0 == fcur, 1.0, 0.0)
                facc = [a * keep for a in facc]
                faccd = faccd * keep + e16
                for r in range(_L):
                    e_b = e16[r]
                    for k in range(128 // _L):
                        facc[k] = facc[k] + e_b * xv[
                            b, g * _L + r, pl.ds(k * _L, _L)]
                return tuple(facc) + (faccd, seg0)

            def slow(*op):
                sacc = list(op[:8])
                saccd = op[8]
                scur = op[9]
                for r in range(_L):
                    seg = b16[r]
                    e_b = e16[r]

                    @pl.when(seg != scur)
                    def _(sacc=sacc, saccd=saccd, scur=scur):
                        flush(jnp.maximum(scur, 0), sacc, saccd)

                    keep = jnp.where(seg == scur, 1.0, 0.0)
                    for k in range(128 // _L):
                        sacc[k] = sacc[k] * keep + e_b * xv[
                            b, g * _L + r, pl.ds(k * _L, _L)]
                    saccd = saccd * keep + e_b * lane0
                    scur = seg
                return tuple(sacc) + (saccd, scur)

            return lax.cond(seg0 == b16[_L - 1], fast, slow, *gc)

        return lax.fori_loop(glo, _CH // _L, group_body, carry)

    carry0 = tuple(zero for _ in range(9)) + (jnp.int32(-1),)
    fc = lax.fori_loop(0, cnt, chunk_body, carry0)
    flush(jnp.maximum(fc[9], 0), list(fc[:8]), fc[8])

    pltpu.sync_copy(tab, feat_hbm.at[wid])
    pltpu.sync_copy(dtab, den_hbm.at[wid])


def _k2(x, lg, batch, m):
    mesh = plsc.VectorSubcoreMesh(core_axis_name="c", subcore_axis_name="s")
    f = pl.kernel(
        _k2_body,
        out_type=[
            jax.ShapeDtypeStruct((_RG * _CG, _TR, 128), jnp.float32),
            jax.ShapeDtypeStruct((_RG * _CG, 64, 128), jnp.float32),
        ],
        mesh=mesh,
        compiler_params=pltpu.CompilerParams(needs_layout_passes=False),
        scratch_types=[
            pltpu.VMEM((2, _CH, 128), jnp.float32),
            pltpu.VMEM((2, _CH), jnp.float32),
            pltpu.VMEM((2, _CH), jnp.int32),
            pltpu.VMEM((_NSEG,), jnp.float32),
            pltpu.VMEM((_TR, 128), jnp.float32),
            pltpu.VMEM((64, 128), jnp.float32),
            pltpu.SemaphoreType.DMA((2,)),
        ],
    )
    return f(x, lg, batch, m)


def _k3_body(p_ref, d_ref, out_ref):
    p = p_ref[...][:, :, :_NSEG, :]
    psum = jnp.sum(p, axis=0)
    feat = jnp.concatenate([psum[g] for g in range(_CG)], axis=1)
    dsum = jnp.sum(d_ref[...], axis=0)
    srow = lax.broadcasted_iota(jnp.int32, (_NSEG, 64), 0)
    rcol = lax.broadcasted_iota(jnp.int32, (_NSEG, 64), 1)
    sel = (rcol == lax.shift_right_logical(srow, 3)).astype(jnp.float32)
    g = jnp.dot(sel, dsum, preferred_element_type=jnp.float32)
    sc = lax.broadcasted_iota(jnp.int32, (_NSEG, 128), 0)
    cc = lax.broadcasted_iota(jnp.int32, (_NSEG, 128), 1)
    win = (lax.shift_right_logical(cc, 4) ==
           lax.bitwise_and(sc, 7)).astype(jnp.float32)
    den = jnp.sum(g * win, axis=1, keepdims=True) * 0.25
    out_ref[...] = feat / (den + 1e-16)


def _k3(p4, d4):
    return pl.pallas_call(
        _k3_body,
        in_specs=[
            pl.BlockSpec((_RG, _CG, _TR, 128), lambda: (0, 0, 0, 0)),
            pl.BlockSpec((_RG * _CG, 64, 128), lambda: (0, 0, 0)),
        ],
        out_specs=pl.BlockSpec((_NSEG, 512), lambda: (0, 0)),
        out_shape=jax.ShapeDtypeStruct((_NSEG, 512), jnp.float32),
    )(p4, d4)


def kernel(x, W1, b1, W2, b2, batch):
    n, d = x.shape
    h = W1.shape[1]
    nseg = _NSEG
    blk = 4000
    nb = n // blk

    batch_i = batch.astype(jnp.int32)
    batch3 = batch_i.reshape(nb, 1, blk)
    b1r = b1.reshape(1, h)

    lg3, m = _k1(x, W1, b1r, W2, batch3, nseg, blk, nb, d, h)
    feat, den = _k2(x, lg3.reshape(n), batch_i, m.reshape(nseg))
    return _k3(feat.reshape(_RG, _CG, _TR, 128), den)
